# baseline (speedup 1.0000x reference)
import functools

import jax
import jax.numpy as jnp
from jax import lax
from jax.experimental import pallas as pl
from jax.experimental.pallas import tpu as pltpu
from jax.experimental.pallas import tpu_sc as plsc

N = 10000
E = 320000
D = 128

NC = 2
NS = 16
L = 16
NW = NC * NS
FH = D // NC
SUB = 128
IDR = 8
CHE = IDR * SUB
NCH = 20
EPT = CHE * NCH
IRT = EPT // SUB
EP = EPT * NS
NP = 10240
RPT = NP // NS
BM = 1000

S2 = 64
SB2 = 8
CH2 = S2 * SB2
EPT2 = EP // NW
NCH2 = EPT2 // CH2
IRT2 = EPT2 // S2

_mesh = plsc.VectorSubcoreMesh(core_axis_name="c", subcore_axis_name="s")
_sc_params = pltpu.CompilerParams(needs_layout_passes=False,
                                  use_tc_tiling_on_sc=False)


def _zero_acc(zb, acc_sh, sid):
    @pl.loop(0, SUB)
    def _z(i):
        for j in range(FH // L):
            zb[i, pl.ds(j * L, L)] = jnp.zeros((L,), jnp.float32)

    for kk in range(RPT // SUB):
        pltpu.sync_copy(zb, acc_sh.at[pl.ds(sid * RPT + kk * SUB, SUB)])


NB = 4


def _sc_body(row_hbm, col_hbm, h_hbm, out_hbm,
             rowi_v, coli_v, wvs, rows_v, acc_sh, isem, gsem, ssem,
             sid, cid, pre_fn, weight_fn, tail_fn):
    hsrc = h_hbm.at[cid]

    def idx_start(c, b):
        r0 = sid * IRT + c * IDR
        pltpu.async_copy(row_hbm.at[pl.ds(r0, IDR)], rowi_v.at[b],
                         isem.at[b])
        pltpu.async_copy(col_hbm.at[pl.ds(r0, IDR)], coli_v.at[b],
                         isem.at[b])

    def idx_wait(c, b):
        r0 = sid * IRT + c * IDR
        pltpu.make_async_copy(
            row_hbm.at[pl.ds(r0, IDR)], rowi_v.at[b], isem.at[b]).wait()
        pltpu.make_async_copy(
            col_hbm.at[pl.ds(r0, IDR)], coli_v.at[b], isem.at[b]).wait()

    def gather_start(cb, s, b):
        pltpu.async_copy(hsrc.at[coli_v.at[cb, s]], rows_v.at[b], gsem.at[b])

    def gather_wait(cb, s, b):
        pltpu.make_async_copy(
            hsrc.at[coli_v.at[cb, s]], rows_v.at[b], gsem.at[b]).wait()

    def scatter_start(cb, s, b):
        pltpu.async_copy(rows_v.at[b], acc_sh.at[rowi_v.at[cb, s]],
                         ssem.at[b], add=True)

    def scatter_drain(b):
        pltpu.make_async_copy(rows_v.at[b], acc_sh.at[pl.ds(0, SUB)],
                              ssem.at[b]).wait()

    def chunk(c, cb):
        pre_fn(c, cb)

        @pl.when(c + 1 < NCH)
        def _():
            idx_start(c + 1, 1 - cb)

        for s in range(IDR):
            b = s % NB
            ahead = s + NB - 1
            ba = ahead % NB
            if ahead < IDR:
                if s == 0:
                    @pl.when(c > 0)
                    def _():
                        scatter_drain(ba)
                else:
                    scatter_drain(ba)
                gather_start(cb, ahead, ba)
            else:
                if ahead == IDR:
                    @pl.when(c + 1 < NCH)
                    def _():
                        idx_wait(c + 1, 1 - cb)

                @pl.when(c + 1 < NCH)
                def _():
                    scatter_drain(ba)
                    gather_start(1 - cb, ahead - IDR, ba)
            weight_fn(cb, s)
            gather_wait(cb, s, b)
            rows_b = rows_v.at[b]

            @pl.loop(0, SUB, unroll=4)
            def _scale(e, s=s, wv=wvs[cb], rows_b=rows_b):
                wb = plsc.load_gather(
                    wv, [jnp.zeros((L,), jnp.int32) + (s * SUB + e)])
                for j in range(FH // L):
                    sl = pl.ds(j * L, L)
                    rows_b[e, sl] = rows_b[e, sl] * wb

            scatter_start(cb, s, b)
        tail_fn(c, cb)

    idx_start(0, 0)
    idx_wait(0, 0)
    for s0 in range(NB - 1):
        gather_start(0, s0, s0)

    @pl.loop(0, NCH, step=2)
    def _main(c):
        chunk(c, 0)
        chunk(c + 1, 1)

    for b in range(NB):
        scatter_drain(b)

    plsc.subcore_barrier()
    pltpu.sync_copy(acc_sh.at[pl.ds(sid * RPT, RPT)],
                    out_hbm.at[cid, pl.ds(sid * RPT, RPT)])


@functools.partial(
    pl.kernel,
    out_type=(
        jax.ShapeDtypeStruct((NC, NP, FH), jnp.float32),
        jax.ShapeDtypeStruct((EP,), jnp.float32),
    ),
    mesh=_mesh,
    compiler_params=_sc_params,
    scratch_types=[
        pltpu.VMEM((2, IDR, SUB), jnp.int32),
        pltpu.VMEM((2, IDR, SUB), jnp.int32),
        pltpu.VMEM((CHE,), jnp.float32),
        pltpu.VMEM((CHE,), jnp.float32),
        pltpu.VMEM((CHE,), jnp.float32),
        pltpu.VMEM((CHE,), jnp.float32),
        pltpu.VMEM((NB, SUB, FH), jnp.float32),
        pltpu.VMEM((N,), jnp.float32),
        pltpu.VMEM((N,), jnp.float32),
        pltpu.VMEM_SHARED((NP, FH), jnp.float32),
        pltpu.SemaphoreType.DMA((2,)),
        pltpu.SemaphoreType.DMA((NB,)),
        pltpu.SemaphoreType.DMA((2,)),
        pltpu.SemaphoreType.DMA((NB,)),
    ],
)
def _sc_attend_agg(row_hbm, col_hbm, adj_hbm, h_hbm, stop_hbm, sbot_hbm,
                   out_hbm, w_hbm,
                   rowi_v, coli_v, adj0_v, adj1_v, w0_v, w1_v, rows_v,
                   stop_v, sbot_v,
                   acc_sh, isem, gsem, asem, ssem):
    cid = lax.axis_index("c")
    sid = lax.axis_index("s")
    advs = (adj0_v, adj1_v)
    wvs = (w0_v, w1_v)

    pltpu.sync_copy(stop_hbm, stop_v)
    pltpu.sync_copy(sbot_hbm, sbot_v)
    _zero_acc(rows_v.at[0], acc_sh, sid)
    plsc.subcore_barrier()

    def adj_start(c, b):
        e0 = sid * EPT + c * CHE
        pltpu.async_copy(adj_hbm.at[pl.ds(e0, CHE)], advs[b], asem.at[b])

    def pre_fn(c, cb):
        e0 = sid * EPT + c * CHE
        pltpu.make_async_copy(
            adj_hbm.at[pl.ds(e0, CHE)], advs[cb], asem.at[cb]).wait()

    def weight_fn(cb, s):
        @pl.loop(0, SUB // L, unroll=2)
        def _wg(g, cb=cb, s=s):
            o = g * L
            r = rowi_v[cb, s, pl.ds(o, L)]
            cc = coli_v[cb, s, pl.ds(o, L)]
            t = (plsc.load_gather(stop_v, [r])
                 + plsc.load_gather(sbot_v, [cc]))
            t = jnp.where(t >= 0.0, t, 0.2 * t)
            w = 1.0 / (1.0 + jnp.exp(-t))
            wvs[cb][pl.ds(s * SUB + o, L)] = (
                w * advs[cb][pl.ds(s * SUB + o, L)])

    def tail_fn(c, cb):
        @pl.when(c + 1 < NCH)
        def _():
            adj_start(c + 1, 1 - cb)

        @pl.when(cid == 0)
        def _():
            e0 = sid * EPT + c * CHE
            pltpu.sync_copy(wvs[cb], w_hbm.at[pl.ds(e0, CHE)])

    adj_start(0, 0)
    _sc_body(row_hbm, col_hbm, h_hbm, out_hbm,
             rowi_v, coli_v, wvs, rows_v, acc_sh, isem, gsem, ssem,
             sid, cid, pre_fn, weight_fn, tail_fn)


@functools.partial(
    pl.kernel,
    out_type=jax.ShapeDtypeStruct((NC, NP, FH), jnp.float32),
    mesh=_mesh,
    compiler_params=_sc_params,
    scratch_types=[
        pltpu.VMEM((2, IDR, SUB), jnp.int32),
        pltpu.VMEM((2, IDR, SUB), jnp.int32),
        pltpu.VMEM((CHE,), jnp.float32),
        pltpu.VMEM((CHE,), jnp.float32),
        pltpu.VMEM((NB, SUB, FH), jnp.float32),
        pltpu.VMEM_SHARED((NP, FH), jnp.float32),
        pltpu.SemaphoreType.DMA((2,)),
        pltpu.SemaphoreType.DMA((NB,)),
        pltpu.SemaphoreType.DMA((2,)),
        pltpu.SemaphoreType.DMA((NB,)),
    ],
)
def _sc_agg(row_hbm, col_hbm, w_hbm, h_hbm,
            out_hbm,
            rowi_v, coli_v, w0_v, w1_v, rows_v, acc_sh,
            isem, gsem, asem, ssem):
    cid = lax.axis_index("c")
    sid = lax.axis_index("s")
    wvs = (w0_v, w1_v)

    _zero_acc(rows_v.at[0], acc_sh, sid)
    plsc.subcore_barrier()

    def w_start(c, b):
        e0 = sid * EPT + c * CHE
        pltpu.async_copy(w_hbm.at[pl.ds(e0, CHE)], wvs[b], asem.at[b])

    def pre_fn(c, cb):
        e0 = sid * EPT + c * CHE
        pltpu.make_async_copy(
            w_hbm.at[pl.ds(e0, CHE)], wvs[cb], asem.at[cb]).wait()

    def weight_fn(cb, s):
        del cb, s

    def tail_fn(c, cb):
        @pl.when(c + 1 < NCH)
        def _():
            w_start(c + 1, 1 - cb)

    w_start(0, 0)
    _sc_body(row_hbm, col_hbm, h_hbm, out_hbm,
             rowi_v, coli_v, wvs, rows_v, acc_sh, isem, gsem, ssem,
             sid, cid, pre_fn, weight_fn, tail_fn)


@functools.partial(
    pl.kernel,
    out_type=jax.ShapeDtypeStruct((NC, NP, D), jnp.float32),
    mesh=_mesh,
    compiler_params=_sc_params,
    scratch_types=[
        pltpu.VMEM((2, SB2, S2), jnp.int32),
        pltpu.VMEM((2, SB2, S2), jnp.int32),
        pltpu.VMEM((CH2,), jnp.float32),
        pltpu.VMEM((CH2,), jnp.float32),
        pltpu.VMEM((NB, S2, D), jnp.float32),
        pltpu.VMEM_SHARED((NP, D), jnp.float32),
        pltpu.SemaphoreType.DMA((2,)),
        pltpu.SemaphoreType.DMA((NB,)),
        pltpu.SemaphoreType.DMA((2,)),
        pltpu.SemaphoreType.DMA((NB,)),
    ],
)
def _sc_agg_full(row_hbm, col_hbm, w_hbm, h_hbm,
                 out_hbm,
                 rowi_v, coli_v, w0_v, w1_v, rows_v, acc_sh,
                 isem, gsem, asem, ssem):
    cid = lax.axis_index("c")
    sid = lax.axis_index("s")
    tid = cid * NS + sid
    wvs = (w0_v, w1_v)

    zb = rows_v.at[0]

    @pl.loop(0, S2)
    def _z(i):
        for j in range(D // L):
            zb[i, pl.ds(j * L, L)] = jnp.zeros((L,), jnp.float32)

    for kk in range(RPT // S2):
        pltpu.sync_copy(zb, acc_sh.at[pl.ds(sid * RPT + kk * S2, S2)])
    plsc.subcore_barrier()

    def idx_start(c, b):
        r0 = tid * IRT2 + c * SB2
        pltpu.async_copy(row_hbm.at[pl.ds(r0, SB2)], rowi_v.at[b],
                         isem.at[b])
        pltpu.async_copy(col_hbm.at[pl.ds(r0, SB2)], coli_v.at[b],
                         isem.at[b])

    def idx_wait(c, b):
        r0 = tid * IRT2 + c * SB2
        pltpu.make_async_copy(
            row_hbm.at[pl.ds(r0, SB2)], rowi_v.at[b], isem.at[b]).wait()
        pltpu.make_async_copy(
            col_hbm.at[pl.ds(r0, SB2)], coli_v.at[b], isem.at[b]).wait()

    def w_start(c, b):
        e0 = tid * EPT2 + c * CH2
        pltpu.async_copy(w_hbm.at[pl.ds(e0, CH2)], wvs[b], asem.at[b])

    def w_wait(c, b):
        e0 = tid * EPT2 + c * CH2
        pltpu.make_async_copy(
            w_hbm.at[pl.ds(e0, CH2)], wvs[b], asem.at[b]).wait()

    def gather_start(cb, s, b):
        pltpu.async_copy(h_hbm.at[coli_v.at[cb, s]], rows_v.at[b],
                         gsem.at[b])

    def gather_wait(cb, s, b):
        pltpu.make_async_copy(
            h_hbm.at[coli_v.at[cb, s]], rows_v.at[b], gsem.at[b]).wait()

    def scatter_start(cb, s, b):
        pltpu.async_copy(rows_v.at[b], acc_sh.at[rowi_v.at[cb, s]],
                         ssem.at[b], add=True)

    def scatter_drain(b):
        pltpu.make_async_copy(rows_v.at[b], acc_sh.at[pl.ds(0, S2)],
                              ssem.at[b]).wait()

    def chunk(c, cb):
        w_wait(c, cb)

        @pl.when(c + 1 < NCH2)
        def _():
            idx_start(c + 1, 1 - cb)
            w_start(c + 1, 1 - cb)

        for s in range(SB2):
            b = s % NB
            ahead = s + NB - 1
            ba = ahead % NB
            if ahead < SB2:
                if s == 0:
                    @pl.when(c > 0)
                    def _():
                        scatter_drain(ba)
                else:
                    scatter_drain(ba)
                gather_start(cb, ahead, ba)
            else:
                if ahead == SB2:
                    @pl.when(c + 1 < NCH2)
                    def _():
                        idx_wait(c + 1, 1 - cb)

                @pl.when(c + 1 < NCH2)
                def _():
                    scatter_drain(ba)
                    gather_start(1 - cb, ahead - SB2, ba)
            gather_wait(cb, s, b)
            rows_b = rows_v.at[b]

            @pl.loop(0, S2, unroll=2)
            def _scale(e, s=s, wv=wvs[cb], rows_b=rows_b):
                wb = plsc.load_gather(
                    wv, [jnp.zeros((L,), jnp.int32) + (s * S2 + e)])
                for j in range(D // L):
                    sl = pl.ds(j * L, L)
                    rows_b[e, sl] = rows_b[e, sl] * wb

            scatter_start(cb, s, b)

    idx_start(0, 0)
    w_start(0, 0)
    idx_wait(0, 0)
    for s0 in range(NB - 1):
        gather_start(0, s0, s0)

    @pl.loop(0, NCH2, step=2)
    def _main(c):
        chunk(c, 0)
        chunk(c + 1, 1)

    for b in range(NB):
        scatter_drain(b)

    plsc.subcore_barrier()
    pltpu.sync_copy(acc_sh.at[pl.ds(sid * RPT, RPT)],
                    out_hbm.at[cid, pl.ds(sid * RPT, RPT)])


def _mm1_body(x_ref, w1_ref, a2_ref, h1_ref, s2_ref):
    h1 = jnp.dot(x_ref[...], w1_ref[...], preferred_element_type=jnp.float32)
    h1_ref[0] = h1[:, :FH]
    h1_ref[1] = h1[:, FH:]
    s2_ref[...] = jnp.dot(h1, a2_ref[...], preferred_element_type=jnp.float32)


_mm1 = pl.pallas_call(
    _mm1_body,
    grid=(N // BM,),
    in_specs=[
        pl.BlockSpec((BM, D), lambda i: (i, 0)),
        pl.BlockSpec((D, D), lambda i: (0, 0)),
        pl.BlockSpec((D, 8), lambda i: (0, 0)),
    ],
    out_specs=[
        pl.BlockSpec((NC, BM, FH), lambda i: (0, i, 0)),
        pl.BlockSpec((BM, 8), lambda i: (i, 0)),
    ],
    out_shape=[
        jax.ShapeDtypeStruct((NC, N, FH), jnp.float32),
        jax.ShapeDtypeStruct((N, 8), jnp.float32),
    ],
)


def _mm2_body(p_ref, w2_ref, h2_ref):
    h = jnp.maximum(jnp.concatenate([p_ref[0], p_ref[1]], axis=1), 0.0)
    h2_ref[...] = jnp.dot(h, w2_ref[...], preferred_element_type=jnp.float32)


_mm2 = pl.pallas_call(
    _mm2_body,
    grid=(N // BM,),
    in_specs=[
        pl.BlockSpec((NC, BM, FH), lambda i: (0, i, 0)),
        pl.BlockSpec((D, D), lambda i: (0, 0)),
    ],
    out_specs=pl.BlockSpec((BM, D), lambda i: (i, 0)),
    out_shape=jax.ShapeDtypeStruct((N, D), jnp.float32),
)


def _final_body(p_ref, x_ref, lnw_ref, lnb_ref, o_ref):
    h = jnp.maximum(p_ref[0] + p_ref[1], 0.0)
    h = h + x_ref[...]
    mean = jnp.mean(h, axis=1, keepdims=True)
    d = h - mean
    var = jnp.mean(d * d, axis=1, keepdims=True)
    o_ref[...] = d * lax.rsqrt(var + 1e-5) * lnw_ref[...] + lnb_ref[...]


_final = pl.pallas_call(
    _final_body,
    grid=(N // BM,),
    in_specs=[
        pl.BlockSpec((NC, BM, D), lambda i: (0, i, 0)),
        pl.BlockSpec((BM, D), lambda i: (i, 0)),
        pl.BlockSpec((1, D), lambda i: (0, 0)),
        pl.BlockSpec((1, D), lambda i: (0, 0)),
    ],
    out_specs=pl.BlockSpec((BM, D), lambda i: (i, 0)),
    out_shape=jax.ShapeDtypeStruct((N, D), jnp.float32),
)


def kernel(x, edge_index, adj_vals, W1, a1, W2, ln_w, ln_b):
    pad = EP - E
    rowp = jnp.concatenate([edge_index[0], jnp.zeros((pad,), jnp.int32)])
    colp = jnp.concatenate([edge_index[1], jnp.zeros((pad,), jnp.int32)])
    row2d = rowp.reshape(EP // SUB, SUB)
    col2d = colp.reshape(EP // SUB, SUB)
    row2d_64 = rowp.reshape(EP // S2, S2)
    col2d_64 = colp.reshape(EP // S2, S2)
    adjp = jnp.concatenate([adj_vals, jnp.zeros((pad,), jnp.float32)])
    a2 = jnp.concatenate([a1[:D], a1[D:]], axis=1)
    a2 = jnp.pad(a2, ((0, 0), (0, 6)))

    h1, s2 = _mm1(x, W1, a2)
    stop = s2[:, 0]
    sbot = s2[:, 1]

    part1, w = _sc_attend_agg(row2d, col2d, adjp, h1, stop, sbot)
    h2 = _mm2(part1, W2)
    part2 = _sc_agg_full(row2d_64, col2d_64, w, h2)
    return _final(part2, x, ln_w.reshape(1, D), ln_b.reshape(1, D))


# speedup vs baseline: 1.2080x; 1.2080x over previous
import functools

import jax
import jax.numpy as jnp
from jax import lax
from jax.experimental import pallas as pl
from jax.experimental.pallas import tpu as pltpu
from jax.experimental.pallas import tpu_sc as plsc

N = 10000
E = 320000
D = 128

NC = 2
NS = 16
L = 16
NW = NC * NS
FH = D // NC
SUB = 128
IDR = 8
CHE = IDR * SUB
NCH = 20
EPT = CHE * NCH
IRT = EPT // SUB
EP = EPT * NS
NP = 10240
RPT = NP // NS
BM = 1000

S2 = 64
SB2 = 8
CH2 = S2 * SB2
EPT2 = EP // NW
NCH2 = EPT2 // CH2
IRT2 = EPT2 // S2

_mesh = plsc.VectorSubcoreMesh(core_axis_name="c", subcore_axis_name="s")
_sc_params = pltpu.CompilerParams(needs_layout_passes=False,
                                  use_tc_tiling_on_sc=False)


def _zero_acc(zb, acc_sh, sid):
    @pl.loop(0, SUB)
    def _z(i):
        for j in range(FH // L):
            zb[i, pl.ds(j * L, L)] = jnp.zeros((L,), jnp.float32)

    for kk in range(RPT // SUB):
        pltpu.sync_copy(zb, acc_sh.at[pl.ds(sid * RPT + kk * SUB, SUB)])


NB = 4


def _sc_body(row_hbm, col_hbm, hsrc, out_hbm,
             rowi_v, coli_v, wvs, rows_v, acc_sh, isem, gsem, ssem,
             sid, cid, pre_fn, weight_fn, tail_fn):
    def idx_start(c, b):
        r0 = sid * IRT + c * IDR
        pltpu.async_copy(row_hbm.at[pl.ds(r0, IDR)], rowi_v.at[b],
                         isem.at[b])
        pltpu.async_copy(col_hbm.at[pl.ds(r0, IDR)], coli_v.at[b],
                         isem.at[b])

    def idx_wait(c, b):
        r0 = sid * IRT + c * IDR
        pltpu.make_async_copy(
            row_hbm.at[pl.ds(r0, IDR)], rowi_v.at[b], isem.at[b]).wait()
        pltpu.make_async_copy(
            col_hbm.at[pl.ds(r0, IDR)], coli_v.at[b], isem.at[b]).wait()

    def gather_start(cb, s, b):
        pltpu.async_copy(hsrc.at[coli_v.at[cb, s]], rows_v.at[b], gsem.at[b])

    def gather_wait(cb, s, b):
        pltpu.make_async_copy(
            hsrc.at[coli_v.at[cb, s]], rows_v.at[b], gsem.at[b]).wait()

    def scatter_start(cb, s, b):
        pltpu.async_copy(rows_v.at[b], acc_sh.at[rowi_v.at[cb, s]],
                         ssem.at[b], add=True)

    def scatter_drain(b):
        pltpu.make_async_copy(rows_v.at[b], acc_sh.at[pl.ds(0, SUB)],
                              ssem.at[b]).wait()

    def chunk(c, cb):
        pre_fn(c, cb)

        @pl.when(c + 1 < NCH)
        def _():
            idx_start(c + 1, 1 - cb)

        for s in range(IDR):
            b = s % NB
            ahead = s + NB - 1
            ba = ahead % NB
            if ahead < IDR:
                if s == 0:
                    @pl.when(c > 0)
                    def _():
                        scatter_drain(ba)
                else:
                    scatter_drain(ba)
                gather_start(cb, ahead, ba)
            else:
                if ahead == IDR:
                    @pl.when(c + 1 < NCH)
                    def _():
                        idx_wait(c + 1, 1 - cb)

                @pl.when(c + 1 < NCH)
                def _():
                    scatter_drain(ba)
                    gather_start(1 - cb, ahead - IDR, ba)
            weight_fn(cb, s)
            gather_wait(cb, s, b)
            rows_b = rows_v.at[b]

            @pl.loop(0, SUB, unroll=4)
            def _scale(e, s=s, wv=wvs[cb], rows_b=rows_b):
                wb = plsc.load_gather(
                    wv, [jnp.zeros((L,), jnp.int32) + (s * SUB + e)])
                for j in range(FH // L):
                    sl = pl.ds(j * L, L)
                    rows_b[e, sl] = rows_b[e, sl] * wb

            scatter_start(cb, s, b)
        tail_fn(c, cb)

    idx_start(0, 0)
    idx_wait(0, 0)
    for s0 in range(NB - 1):
        gather_start(0, s0, s0)

    @pl.loop(0, NCH, step=2)
    def _main(c):
        chunk(c, 0)
        chunk(c + 1, 1)

    for b in range(NB):
        scatter_drain(b)

    plsc.subcore_barrier()
    pltpu.sync_copy(acc_sh.at[pl.ds(sid * RPT, RPT)],
                    out_hbm.at[cid, pl.ds(sid * RPT, RPT)])


@functools.partial(
    pl.kernel,
    out_type=(
        jax.ShapeDtypeStruct((NC, NP, FH), jnp.float32),
        jax.ShapeDtypeStruct((EP,), jnp.float32),
    ),
    mesh=_mesh,
    compiler_params=_sc_params,
    scratch_types=[
        pltpu.VMEM((2, IDR, SUB), jnp.int32),
        pltpu.VMEM((2, IDR, SUB), jnp.int32),
        pltpu.VMEM((CHE,), jnp.float32),
        pltpu.VMEM((CHE,), jnp.float32),
        pltpu.VMEM((CHE,), jnp.float32),
        pltpu.VMEM((CHE,), jnp.float32),
        pltpu.VMEM((NB, SUB, FH), jnp.float32),
        pltpu.VMEM((N,), jnp.float32),
        pltpu.VMEM((N,), jnp.float32),
        pltpu.VMEM_SHARED((NP, FH), jnp.float32),
        pltpu.SemaphoreType.DMA((2,)),
        pltpu.SemaphoreType.DMA((NB,)),
        pltpu.SemaphoreType.DMA((2,)),
        pltpu.SemaphoreType.DMA((NB,)),
    ],
)
def _sc_attend_agg(row_hbm, col_hbm, adj_hbm, h_hbm, stop_hbm, sbot_hbm,
                   out_hbm, w_hbm,
                   rowi_v, coli_v, adj0_v, adj1_v, w0_v, w1_v, rows_v,
                   stop_v, sbot_v,
                   acc_sh, isem, gsem, asem, ssem):
    cid = lax.axis_index("c")
    sid = lax.axis_index("s")
    advs = (adj0_v, adj1_v)
    wvs = (w0_v, w1_v)

    pltpu.sync_copy(stop_hbm, stop_v)
    pltpu.sync_copy(sbot_hbm, sbot_v)
    _zero_acc(rows_v.at[0], acc_sh, sid)
    plsc.subcore_barrier()

    def adj_start(c, b):
        e0 = sid * EPT + c * CHE
        pltpu.async_copy(adj_hbm.at[pl.ds(e0, CHE)], advs[b], asem.at[b])

    def pre_fn(c, cb):
        e0 = sid * EPT + c * CHE
        pltpu.make_async_copy(
            adj_hbm.at[pl.ds(e0, CHE)], advs[cb], asem.at[cb]).wait()

    def weight_fn(cb, s):
        @pl.loop(0, SUB // L, unroll=2)
        def _wg(g, cb=cb, s=s):
            o = g * L
            r = rowi_v[cb, s, pl.ds(o, L)]
            cc = coli_v[cb, s, pl.ds(o, L)]
            t = (plsc.load_gather(stop_v, [r])
                 + plsc.load_gather(sbot_v, [cc]))
            t = jnp.where(t >= 0.0, t, 0.2 * t)
            w = 1.0 / (1.0 + jnp.exp(-t))
            wvs[cb][pl.ds(s * SUB + o, L)] = (
                w * advs[cb][pl.ds(s * SUB + o, L)])

    def tail_fn(c, cb):
        @pl.when(c + 1 < NCH)
        def _():
            adj_start(c + 1, 1 - cb)

        @pl.when(cid == 0)
        def _():
            e0 = sid * EPT + c * CHE
            pltpu.sync_copy(wvs[cb], w_hbm.at[pl.ds(e0, CHE)])

    adj_start(0, 0)
    _sc_body(row_hbm, col_hbm, h_hbm.at[cid], out_hbm,
             rowi_v, coli_v, wvs, rows_v, acc_sh, isem, gsem, ssem,
             sid, cid, pre_fn, weight_fn, tail_fn)


@functools.partial(
    pl.kernel,
    out_type=jax.ShapeDtypeStruct((NC, NP, FH), jnp.float32),
    mesh=_mesh,
    compiler_params=_sc_params,
    scratch_types=[
        pltpu.VMEM((2, IDR, SUB), jnp.int32),
        pltpu.VMEM((2, IDR, SUB), jnp.int32),
        pltpu.VMEM((CHE,), jnp.float32),
        pltpu.VMEM((CHE,), jnp.float32),
        pltpu.VMEM((NB, SUB, FH), jnp.float32),
        pltpu.VMEM_SHARED((NP, FH), jnp.float32),
        pltpu.VMEM_SHARED((N, FH), jnp.float32),
        pltpu.SemaphoreType.DMA((2,)),
        pltpu.SemaphoreType.DMA((NB,)),
        pltpu.SemaphoreType.DMA((2,)),
        pltpu.SemaphoreType.DMA((NB,)),
    ],
)
def _sc_agg(row_hbm, col_hbm, w_hbm, h_hbm,
            out_hbm,
            rowi_v, coli_v, w0_v, w1_v, rows_v, acc_sh, h_sh,
            isem, gsem, asem, ssem):
    cid = lax.axis_index("c")
    sid = lax.axis_index("s")
    wvs = (w0_v, w1_v)

    _zero_acc(rows_v.at[0], acc_sh, sid)
    pltpu.sync_copy(h_hbm.at[cid, pl.ds(sid * (N // NS), N // NS)],
                    h_sh.at[pl.ds(sid * (N // NS), N // NS)])
    plsc.subcore_barrier()

    def w_start(c, b):
        e0 = sid * EPT + c * CHE
        pltpu.async_copy(w_hbm.at[pl.ds(e0, CHE)], wvs[b], asem.at[b])

    def pre_fn(c, cb):
        e0 = sid * EPT + c * CHE
        pltpu.make_async_copy(
            w_hbm.at[pl.ds(e0, CHE)], wvs[cb], asem.at[cb]).wait()

    def weight_fn(cb, s):
        del cb, s

    def tail_fn(c, cb):
        @pl.when(c + 1 < NCH)
        def _():
            w_start(c + 1, 1 - cb)

    w_start(0, 0)
    _sc_body(row_hbm, col_hbm, h_sh, out_hbm,
             rowi_v, coli_v, wvs, rows_v, acc_sh, isem, gsem, ssem,
             sid, cid, pre_fn, weight_fn, tail_fn)


@functools.partial(
    pl.kernel,
    out_type=jax.ShapeDtypeStruct((NC, NP, D), jnp.float32),
    mesh=_mesh,
    compiler_params=_sc_params,
    scratch_types=[
        pltpu.VMEM((2, SB2, S2), jnp.int32),
        pltpu.VMEM((2, SB2, S2), jnp.int32),
        pltpu.VMEM((CH2,), jnp.float32),
        pltpu.VMEM((CH2,), jnp.float32),
        pltpu.VMEM((NB, S2, D), jnp.float32),
        pltpu.VMEM_SHARED((NP, D), jnp.float32),
        pltpu.SemaphoreType.DMA((2,)),
        pltpu.SemaphoreType.DMA((NB,)),
        pltpu.SemaphoreType.DMA((2,)),
        pltpu.SemaphoreType.DMA((NB,)),
    ],
)
def _sc_agg_full(row_hbm, col_hbm, w_hbm, h_hbm,
                 out_hbm,
                 rowi_v, coli_v, w0_v, w1_v, rows_v, acc_sh,
                 isem, gsem, asem, ssem):
    cid = lax.axis_index("c")
    sid = lax.axis_index("s")
    tid = cid * NS + sid
    wvs = (w0_v, w1_v)

    zb = rows_v.at[0]

    @pl.loop(0, S2)
    def _z(i):
        for j in range(D // L):
            zb[i, pl.ds(j * L, L)] = jnp.zeros((L,), jnp.float32)

    for kk in range(RPT // S2):
        pltpu.sync_copy(zb, acc_sh.at[pl.ds(sid * RPT + kk * S2, S2)])
    plsc.subcore_barrier()

    def idx_start(c, b):
        r0 = tid * IRT2 + c * SB2
        pltpu.async_copy(row_hbm.at[pl.ds(r0, SB2)], rowi_v.at[b],
                         isem.at[b])
        pltpu.async_copy(col_hbm.at[pl.ds(r0, SB2)], coli_v.at[b],
                         isem.at[b])

    def idx_wait(c, b):
        r0 = tid * IRT2 + c * SB2
        pltpu.make_async_copy(
            row_hbm.at[pl.ds(r0, SB2)], rowi_v.at[b], isem.at[b]).wait()
        pltpu.make_async_copy(
            col_hbm.at[pl.ds(r0, SB2)], coli_v.at[b], isem.at[b]).wait()

    def w_start(c, b):
        e0 = tid * EPT2 + c * CH2
        pltpu.async_copy(w_hbm.at[pl.ds(e0, CH2)], wvs[b], asem.at[b])

    def w_wait(c, b):
        e0 = tid * EPT2 + c * CH2
        pltpu.make_async_copy(
            w_hbm.at[pl.ds(e0, CH2)], wvs[b], asem.at[b]).wait()

    def gather_start(cb, s, b):
        pltpu.async_copy(h_hbm.at[coli_v.at[cb, s]], rows_v.at[b],
                         gsem.at[b])

    def gather_wait(cb, s, b):
        pltpu.make_async_copy(
            h_hbm.at[coli_v.at[cb, s]], rows_v.at[b], gsem.at[b]).wait()

    def scatter_start(cb, s, b):
        pltpu.async_copy(rows_v.at[b], acc_sh.at[rowi_v.at[cb, s]],
                         ssem.at[b], add=True)

    def scatter_drain(b):
        pltpu.make_async_copy(rows_v.at[b], acc_sh.at[pl.ds(0, S2)],
                              ssem.at[b]).wait()

    def chunk(c, cb):
        w_wait(c, cb)

        @pl.when(c + 1 < NCH2)
        def _():
            idx_start(c + 1, 1 - cb)
            w_start(c + 1, 1 - cb)

        for s in range(SB2):
            b = s % NB
            ahead = s + NB - 1
            ba = ahead % NB
            if ahead < SB2:
                if s == 0:
                    @pl.when(c > 0)
                    def _():
                        scatter_drain(ba)
                else:
                    scatter_drain(ba)
                gather_start(cb, ahead, ba)
            else:
                if ahead == SB2:
                    @pl.when(c + 1 < NCH2)
                    def _():
                        idx_wait(c + 1, 1 - cb)

                @pl.when(c + 1 < NCH2)
                def _():
                    scatter_drain(ba)
                    gather_start(1 - cb, ahead - SB2, ba)
            gather_wait(cb, s, b)
            rows_b = rows_v.at[b]

            @pl.loop(0, S2, unroll=2)
            def _scale(e, s=s, wv=wvs[cb], rows_b=rows_b):
                wb = plsc.load_gather(
                    wv, [jnp.zeros((L,), jnp.int32) + (s * S2 + e)])
                for j in range(D // L):
                    sl = pl.ds(j * L, L)
                    rows_b[e, sl] = rows_b[e, sl] * wb

            scatter_start(cb, s, b)

    idx_start(0, 0)
    w_start(0, 0)
    idx_wait(0, 0)
    for s0 in range(NB - 1):
        gather_start(0, s0, s0)

    @pl.loop(0, NCH2, step=2)
    def _main(c):
        chunk(c, 0)
        chunk(c + 1, 1)

    for b in range(NB):
        scatter_drain(b)

    plsc.subcore_barrier()
    pltpu.sync_copy(acc_sh.at[pl.ds(sid * RPT, RPT)],
                    out_hbm.at[cid, pl.ds(sid * RPT, RPT)])


def _mm1_body(x_ref, w1_ref, a2_ref, h1_ref, s2_ref):
    h1 = jnp.dot(x_ref[...], w1_ref[...], preferred_element_type=jnp.float32)
    h1_ref[0] = h1[:, :FH]
    h1_ref[1] = h1[:, FH:]
    s2_ref[...] = jnp.dot(h1, a2_ref[...], preferred_element_type=jnp.float32)


_mm1 = pl.pallas_call(
    _mm1_body,
    grid=(N // BM,),
    in_specs=[
        pl.BlockSpec((BM, D), lambda i: (i, 0)),
        pl.BlockSpec((D, D), lambda i: (0, 0)),
        pl.BlockSpec((D, 8), lambda i: (0, 0)),
    ],
    out_specs=[
        pl.BlockSpec((NC, BM, FH), lambda i: (0, i, 0)),
        pl.BlockSpec((BM, 8), lambda i: (i, 0)),
    ],
    out_shape=[
        jax.ShapeDtypeStruct((NC, N, FH), jnp.float32),
        jax.ShapeDtypeStruct((N, 8), jnp.float32),
    ],
)


def _mm2_body(p_ref, w2_ref, h2_ref):
    h = jnp.maximum(jnp.concatenate([p_ref[0], p_ref[1]], axis=1), 0.0)
    h2 = jnp.dot(h, w2_ref[...], preferred_element_type=jnp.float32)
    h2_ref[0] = h2[:, :FH]
    h2_ref[1] = h2[:, FH:]


_mm2 = pl.pallas_call(
    _mm2_body,
    grid=(N // BM,),
    in_specs=[
        pl.BlockSpec((NC, BM, FH), lambda i: (0, i, 0)),
        pl.BlockSpec((D, D), lambda i: (0, 0)),
    ],
    out_specs=pl.BlockSpec((NC, BM, FH), lambda i: (0, i, 0)),
    out_shape=jax.ShapeDtypeStruct((NC, N, FH), jnp.float32),
)


def _final_body(p_ref, x_ref, lnw_ref, lnb_ref, o_ref):
    h = jnp.maximum(jnp.concatenate([p_ref[0], p_ref[1]], axis=1), 0.0)
    h = h + x_ref[...]
    mean = jnp.mean(h, axis=1, keepdims=True)
    d = h - mean
    var = jnp.mean(d * d, axis=1, keepdims=True)
    o_ref[...] = d * lax.rsqrt(var + 1e-5) * lnw_ref[...] + lnb_ref[...]


_final = pl.pallas_call(
    _final_body,
    grid=(N // BM,),
    in_specs=[
        pl.BlockSpec((NC, BM, FH), lambda i: (0, i, 0)),
        pl.BlockSpec((BM, D), lambda i: (i, 0)),
        pl.BlockSpec((1, D), lambda i: (0, 0)),
        pl.BlockSpec((1, D), lambda i: (0, 0)),
    ],
    out_specs=pl.BlockSpec((BM, D), lambda i: (i, 0)),
    out_shape=jax.ShapeDtypeStruct((N, D), jnp.float32),
)


def kernel(x, edge_index, adj_vals, W1, a1, W2, ln_w, ln_b):
    pad = EP - E
    rowp = jnp.concatenate([edge_index[0], jnp.zeros((pad,), jnp.int32)])
    colp = jnp.concatenate([edge_index[1], jnp.zeros((pad,), jnp.int32)])
    row2d = rowp.reshape(EP // SUB, SUB)
    col2d = colp.reshape(EP // SUB, SUB)
    row2d_64 = rowp.reshape(EP // S2, S2)
    col2d_64 = colp.reshape(EP // S2, S2)
    adjp = jnp.concatenate([adj_vals, jnp.zeros((pad,), jnp.float32)])
    a2 = jnp.concatenate([a1[:D], a1[D:]], axis=1)
    a2 = jnp.pad(a2, ((0, 0), (0, 6)))

    h1, s2 = _mm1(x, W1, a2)
    stop = s2[:, 0]
    sbot = s2[:, 1]

    part1, w = _sc_attend_agg(row2d, col2d, adjp, h1, stop, sbot)
    h2 = _mm2(part1, W2)
    part2 = _sc_agg(row2d, col2d, w, h2)
    return _final(part2, x, ln_w.reshape(1, D), ln_b.reshape(1, D))


# speedup vs baseline: 1.4558x; 1.2052x over previous
import functools

import jax
import jax.numpy as jnp
from jax import lax
from jax.experimental import pallas as pl
from jax.experimental.pallas import tpu as pltpu
from jax.experimental.pallas import tpu_sc as plsc

N = 10000
E = 320000
D = 128

NC = 2
NS = 16
L = 16
NW = NC * NS
FH = D // NC
SUB = 128
IDR = 8
CHE = IDR * SUB
NCH = 20
EPT = CHE * NCH
IRT = EPT // SUB
EP = EPT * NS
NP = 10240
RPT = NP // NS
BM = 1000

S2 = 64
SB2 = 8
CH2 = S2 * SB2
EPT2 = EP // NW
NCH2 = EPT2 // CH2
IRT2 = EPT2 // S2

_mesh = plsc.VectorSubcoreMesh(core_axis_name="c", subcore_axis_name="s")
_sc_params = pltpu.CompilerParams(needs_layout_passes=False,
                                  use_tc_tiling_on_sc=False)


def _zero_acc(zb, acc_sh, sid):
    @pl.loop(0, SUB)
    def _z(i):
        for j in range(FH // L):
            zb[i, pl.ds(j * L, L)] = jnp.zeros((L,), jnp.float32)

    for kk in range(RPT // SUB):
        pltpu.sync_copy(zb, acc_sh.at[pl.ds(sid * RPT + kk * SUB, SUB)])


NB = 4


def _sc_body(row_hbm, col_hbm, hsrc, out_hbm,
             rowi_v, coli_v, wvs, rows_v, acc_sh, isem, gsem, ssem,
             sid, cid, pre_fn, weight_fn, tail_fn, nb=NB):
    def idx_start(c, b):
        r0 = sid * IRT + c * IDR
        pltpu.async_copy(row_hbm.at[pl.ds(r0, IDR)], rowi_v.at[b],
                         isem.at[b])
        pltpu.async_copy(col_hbm.at[pl.ds(r0, IDR)], coli_v.at[b],
                         isem.at[b])

    def idx_wait(c, b):
        r0 = sid * IRT + c * IDR
        pltpu.make_async_copy(
            row_hbm.at[pl.ds(r0, IDR)], rowi_v.at[b], isem.at[b]).wait()
        pltpu.make_async_copy(
            col_hbm.at[pl.ds(r0, IDR)], coli_v.at[b], isem.at[b]).wait()

    def gather_start(cb, s, b):
        pltpu.async_copy(hsrc.at[coli_v.at[cb, s]], rows_v.at[b], gsem.at[b])

    def gather_wait(cb, s, b):
        pltpu.make_async_copy(
            hsrc.at[coli_v.at[cb, s]], rows_v.at[b], gsem.at[b]).wait()

    def scatter_start(cb, s, b):
        pltpu.async_copy(rows_v.at[b], acc_sh.at[rowi_v.at[cb, s]],
                         ssem.at[b], add=True)

    def scatter_drain(b):
        pltpu.make_async_copy(rows_v.at[b], acc_sh.at[pl.ds(0, SUB)],
                              ssem.at[b]).wait()

    def chunk(c, cb):
        pre_fn(c, cb)

        @pl.when(c + 1 < NCH)
        def _():
            idx_start(c + 1, 1 - cb)

        for s in range(IDR):
            b = s % nb
            ahead = s + nb - 1
            ba = ahead % nb
            if ahead < IDR:
                if s == 0:
                    @pl.when(c > 0)
                    def _():
                        scatter_drain(ba)
                else:
                    scatter_drain(ba)
                gather_start(cb, ahead, ba)
            else:
                if ahead == IDR:
                    @pl.when(c + 1 < NCH)
                    def _():
                        idx_wait(c + 1, 1 - cb)

                @pl.when(c + 1 < NCH)
                def _():
                    scatter_drain(ba)
                    gather_start(1 - cb, ahead - IDR, ba)
            weight_fn(cb, s)
            gather_wait(cb, s, b)
            rows_b = rows_v.at[b]

            @pl.loop(0, SUB, unroll=4)
            def _scale(e, s=s, wv=wvs[cb], rows_b=rows_b):
                wb = plsc.load_gather(
                    wv, [jnp.zeros((L,), jnp.int32) + (s * SUB + e)])
                for j in range(FH // L):
                    sl = pl.ds(j * L, L)
                    rows_b[e, sl] = rows_b[e, sl] * wb

            scatter_start(cb, s, b)
        tail_fn(c, cb)

    idx_start(0, 0)
    idx_wait(0, 0)
    for s0 in range(nb - 1):
        gather_start(0, s0, s0)

    @pl.loop(0, NCH, step=2)
    def _main(c):
        chunk(c, 0)
        chunk(c + 1, 1)

    for b in range(nb):
        scatter_drain(b)

    plsc.subcore_barrier()
    pltpu.sync_copy(acc_sh.at[pl.ds(sid * RPT, RPT)],
                    out_hbm.at[cid, pl.ds(sid * RPT, RPT)])


@functools.partial(
    pl.kernel,
    out_type=(
        jax.ShapeDtypeStruct((NC, NP, FH), jnp.float32),
        jax.ShapeDtypeStruct((EP,), jnp.float32),
    ),
    mesh=_mesh,
    compiler_params=_sc_params,
    scratch_types=[
        pltpu.VMEM((2, IDR, SUB), jnp.int32),
        pltpu.VMEM((2, IDR, SUB), jnp.int32),
        pltpu.VMEM((CHE,), jnp.float32),
        pltpu.VMEM((CHE,), jnp.float32),
        pltpu.VMEM((CHE,), jnp.float32),
        pltpu.VMEM((CHE,), jnp.float32),
        pltpu.VMEM((2, SUB, FH), jnp.float32),
        pltpu.VMEM((N,), jnp.float32),
        pltpu.VMEM((N,), jnp.float32),
        pltpu.VMEM_SHARED((NP, FH), jnp.float32),
        pltpu.VMEM_SHARED((N, FH), jnp.float32),
        pltpu.SemaphoreType.DMA((2,)),
        pltpu.SemaphoreType.DMA((2,)),
        pltpu.SemaphoreType.DMA((2,)),
        pltpu.SemaphoreType.DMA((2,)),
    ],
)
def _sc_attend_agg(row_hbm, col_hbm, adj_hbm, h_hbm, stop_hbm, sbot_hbm,
                   out_hbm, w_hbm,
                   rowi_v, coli_v, adj0_v, adj1_v, w0_v, w1_v, rows_v,
                   stop_v, sbot_v,
                   acc_sh, h_sh, isem, gsem, asem, ssem):
    cid = lax.axis_index("c")
    sid = lax.axis_index("s")
    advs = (adj0_v, adj1_v)
    wvs = (w0_v, w1_v)

    pltpu.sync_copy(stop_hbm, stop_v)
    pltpu.sync_copy(sbot_hbm, sbot_v)
    _zero_acc(rows_v.at[0], acc_sh, sid)
    pltpu.sync_copy(h_hbm.at[cid, pl.ds(sid * (N // NS), N // NS)],
                    h_sh.at[pl.ds(sid * (N // NS), N // NS)])
    plsc.subcore_barrier()

    def adj_start(c, b):
        e0 = sid * EPT + c * CHE
        pltpu.async_copy(adj_hbm.at[pl.ds(e0, CHE)], advs[b], asem.at[b])

    def pre_fn(c, cb):
        e0 = sid * EPT + c * CHE
        pltpu.make_async_copy(
            adj_hbm.at[pl.ds(e0, CHE)], advs[cb], asem.at[cb]).wait()

    def weight_fn(cb, s):
        @pl.loop(0, SUB // L, unroll=2)
        def _wg(g, cb=cb, s=s):
            o = g * L
            r = rowi_v[cb, s, pl.ds(o, L)]
            cc = coli_v[cb, s, pl.ds(o, L)]
            t = (plsc.load_gather(stop_v, [r])
                 + plsc.load_gather(sbot_v, [cc]))
            t = jnp.where(t >= 0.0, t, 0.2 * t)
            w = 1.0 / (1.0 + jnp.exp(-t))
            wvs[cb][pl.ds(s * SUB + o, L)] = (
                w * advs[cb][pl.ds(s * SUB + o, L)])

    def tail_fn(c, cb):
        @pl.when(c + 1 < NCH)
        def _():
            adj_start(c + 1, 1 - cb)

        @pl.when(cid == 0)
        def _():
            e0 = sid * EPT + c * CHE
            pltpu.sync_copy(wvs[cb], w_hbm.at[pl.ds(e0, CHE)])

    adj_start(0, 0)
    _sc_body(row_hbm, col_hbm, h_sh, out_hbm,
             rowi_v, coli_v, wvs, rows_v, acc_sh, isem, gsem, ssem,
             sid, cid, pre_fn, weight_fn, tail_fn, nb=2)


@functools.partial(
    pl.kernel,
    out_type=jax.ShapeDtypeStruct((NC, NP, FH), jnp.float32),
    mesh=_mesh,
    compiler_params=_sc_params,
    scratch_types=[
        pltpu.VMEM((2, IDR, SUB), jnp.int32),
        pltpu.VMEM((2, IDR, SUB), jnp.int32),
        pltpu.VMEM((CHE,), jnp.float32),
        pltpu.VMEM((CHE,), jnp.float32),
        pltpu.VMEM((NB, SUB, FH), jnp.float32),
        pltpu.VMEM_SHARED((NP, FH), jnp.float32),
        pltpu.VMEM_SHARED((N, FH), jnp.float32),
        pltpu.SemaphoreType.DMA((2,)),
        pltpu.SemaphoreType.DMA((NB,)),
        pltpu.SemaphoreType.DMA((2,)),
        pltpu.SemaphoreType.DMA((NB,)),
    ],
)
def _sc_agg(row_hbm, col_hbm, w_hbm, h_hbm,
            out_hbm,
            rowi_v, coli_v, w0_v, w1_v, rows_v, acc_sh, h_sh,
            isem, gsem, asem, ssem):
    cid = lax.axis_index("c")
    sid = lax.axis_index("s")
    wvs = (w0_v, w1_v)

    _zero_acc(rows_v.at[0], acc_sh, sid)
    pltpu.sync_copy(h_hbm.at[cid, pl.ds(sid * (N // NS), N // NS)],
                    h_sh.at[pl.ds(sid * (N // NS), N // NS)])
    plsc.subcore_barrier()

    def w_start(c, b):
        e0 = sid * EPT + c * CHE
        pltpu.async_copy(w_hbm.at[pl.ds(e0, CHE)], wvs[b], asem.at[b])

    def pre_fn(c, cb):
        e0 = sid * EPT + c * CHE
        pltpu.make_async_copy(
            w_hbm.at[pl.ds(e0, CHE)], wvs[cb], asem.at[cb]).wait()

    def weight_fn(cb, s):
        del cb, s

    def tail_fn(c, cb):
        @pl.when(c + 1 < NCH)
        def _():
            w_start(c + 1, 1 - cb)

    w_start(0, 0)
    _sc_body(row_hbm, col_hbm, h_sh, out_hbm,
             rowi_v, coli_v, wvs, rows_v, acc_sh, isem, gsem, ssem,
             sid, cid, pre_fn, weight_fn, tail_fn)


@functools.partial(
    pl.kernel,
    out_type=jax.ShapeDtypeStruct((NC, NP, D), jnp.float32),
    mesh=_mesh,
    compiler_params=_sc_params,
    scratch_types=[
        pltpu.VMEM((2, SB2, S2), jnp.int32),
        pltpu.VMEM((2, SB2, S2), jnp.int32),
        pltpu.VMEM((CH2,), jnp.float32),
        pltpu.VMEM((CH2,), jnp.float32),
        pltpu.VMEM((NB, S2, D), jnp.float32),
        pltpu.VMEM_SHARED((NP, D), jnp.float32),
        pltpu.SemaphoreType.DMA((2,)),
        pltpu.SemaphoreType.DMA((NB,)),
        pltpu.SemaphoreType.DMA((2,)),
        pltpu.SemaphoreType.DMA((NB,)),
    ],
)
def _sc_agg_full(row_hbm, col_hbm, w_hbm, h_hbm,
                 out_hbm,
                 rowi_v, coli_v, w0_v, w1_v, rows_v, acc_sh,
                 isem, gsem, asem, ssem):
    cid = lax.axis_index("c")
    sid = lax.axis_index("s")
    tid = cid * NS + sid
    wvs = (w0_v, w1_v)

    zb = rows_v.at[0]

    @pl.loop(0, S2)
    def _z(i):
        for j in range(D // L):
            zb[i, pl.ds(j * L, L)] = jnp.zeros((L,), jnp.float32)

    for kk in range(RPT // S2):
        pltpu.sync_copy(zb, acc_sh.at[pl.ds(sid * RPT + kk * S2, S2)])
    plsc.subcore_barrier()

    def idx_start(c, b):
        r0 = tid * IRT2 + c * SB2
        pltpu.async_copy(row_hbm.at[pl.ds(r0, SB2)], rowi_v.at[b],
                         isem.at[b])
        pltpu.async_copy(col_hbm.at[pl.ds(r0, SB2)], coli_v.at[b],
                         isem.at[b])

    def idx_wait(c, b):
        r0 = tid * IRT2 + c * SB2
        pltpu.make_async_copy(
            row_hbm.at[pl.ds(r0, SB2)], rowi_v.at[b], isem.at[b]).wait()
        pltpu.make_async_copy(
            col_hbm.at[pl.ds(r0, SB2)], coli_v.at[b], isem.at[b]).wait()

    def w_start(c, b):
        e0 = tid * EPT2 + c * CH2
        pltpu.async_copy(w_hbm.at[pl.ds(e0, CH2)], wvs[b], asem.at[b])

    def w_wait(c, b):
        e0 = tid * EPT2 + c * CH2
        pltpu.make_async_copy(
            w_hbm.at[pl.ds(e0, CH2)], wvs[b], asem.at[b]).wait()

    def gather_start(cb, s, b):
        pltpu.async_copy(h_hbm.at[coli_v.at[cb, s]], rows_v.at[b],
                         gsem.at[b])

    def gather_wait(cb, s, b):
        pltpu.make_async_copy(
            h_hbm.at[coli_v.at[cb, s]], rows_v.at[b], gsem.at[b]).wait()

    def scatter_start(cb, s, b):
        pltpu.async_copy(rows_v.at[b], acc_sh.at[rowi_v.at[cb, s]],
                         ssem.at[b], add=True)

    def scatter_drain(b):
        pltpu.make_async_copy(rows_v.at[b], acc_sh.at[pl.ds(0, S2)],
                              ssem.at[b]).wait()

    def chunk(c, cb):
        w_wait(c, cb)

        @pl.when(c + 1 < NCH2)
        def _():
            idx_start(c + 1, 1 - cb)
            w_start(c + 1, 1 - cb)

        for s in range(SB2):
            b = s % NB
            ahead = s + NB - 1
            ba = ahead % NB
            if ahead < SB2:
                if s == 0:
                    @pl.when(c > 0)
                    def _():
                        scatter_drain(ba)
                else:
                    scatter_drain(ba)
                gather_start(cb, ahead, ba)
            else:
                if ahead == SB2:
                    @pl.when(c + 1 < NCH2)
                    def _():
                        idx_wait(c + 1, 1 - cb)

                @pl.when(c + 1 < NCH2)
                def _():
                    scatter_drain(ba)
                    gather_start(1 - cb, ahead - SB2, ba)
            gather_wait(cb, s, b)
            rows_b = rows_v.at[b]

            @pl.loop(0, S2, unroll=2)
            def _scale(e, s=s, wv=wvs[cb], rows_b=rows_b):
                wb = plsc.load_gather(
                    wv, [jnp.zeros((L,), jnp.int32) + (s * S2 + e)])
                for j in range(D // L):
                    sl = pl.ds(j * L, L)
                    rows_b[e, sl] = rows_b[e, sl] * wb

            scatter_start(cb, s, b)

    idx_start(0, 0)
    w_start(0, 0)
    idx_wait(0, 0)
    for s0 in range(NB - 1):
        gather_start(0, s0, s0)

    @pl.loop(0, NCH2, step=2)
    def _main(c):
        chunk(c, 0)
        chunk(c + 1, 1)

    for b in range(NB):
        scatter_drain(b)

    plsc.subcore_barrier()
    pltpu.sync_copy(acc_sh.at[pl.ds(sid * RPT, RPT)],
                    out_hbm.at[cid, pl.ds(sid * RPT, RPT)])


def _mm1_body(x_ref, w1_ref, a2_ref, h1_ref, s2_ref):
    h1 = jnp.dot(x_ref[...], w1_ref[...], preferred_element_type=jnp.float32)
    h1_ref[0] = h1[:, :FH]
    h1_ref[1] = h1[:, FH:]
    s2_ref[...] = jnp.dot(h1, a2_ref[...], preferred_element_type=jnp.float32)


_mm1 = pl.pallas_call(
    _mm1_body,
    grid=(N // BM,),
    in_specs=[
        pl.BlockSpec((BM, D), lambda i: (i, 0)),
        pl.BlockSpec((D, D), lambda i: (0, 0)),
        pl.BlockSpec((D, 8), lambda i: (0, 0)),
    ],
    out_specs=[
        pl.BlockSpec((NC, BM, FH), lambda i: (0, i, 0)),
        pl.BlockSpec((BM, 8), lambda i: (i, 0)),
    ],
    out_shape=[
        jax.ShapeDtypeStruct((NC, N, FH), jnp.float32),
        jax.ShapeDtypeStruct((N, 8), jnp.float32),
    ],
)


def _mm2_body(p_ref, w2_ref, h2_ref):
    h = jnp.maximum(jnp.concatenate([p_ref[0], p_ref[1]], axis=1), 0.0)
    h2 = jnp.dot(h, w2_ref[...], preferred_element_type=jnp.float32)
    h2_ref[0] = h2[:, :FH]
    h2_ref[1] = h2[:, FH:]


_mm2 = pl.pallas_call(
    _mm2_body,
    grid=(N // BM,),
    in_specs=[
        pl.BlockSpec((NC, BM, FH), lambda i: (0, i, 0)),
        pl.BlockSpec((D, D), lambda i: (0, 0)),
    ],
    out_specs=pl.BlockSpec((NC, BM, FH), lambda i: (0, i, 0)),
    out_shape=jax.ShapeDtypeStruct((NC, N, FH), jnp.float32),
)


def _final_body(p_ref, x_ref, lnw_ref, lnb_ref, o_ref):
    h = jnp.maximum(jnp.concatenate([p_ref[0], p_ref[1]], axis=1), 0.0)
    h = h + x_ref[...]
    mean = jnp.mean(h, axis=1, keepdims=True)
    d = h - mean
    var = jnp.mean(d * d, axis=1, keepdims=True)
    o_ref[...] = d * lax.rsqrt(var + 1e-5) * lnw_ref[...] + lnb_ref[...]


_final = pl.pallas_call(
    _final_body,
    grid=(N // BM,),
    in_specs=[
        pl.BlockSpec((NC, BM, FH), lambda i: (0, i, 0)),
        pl.BlockSpec((BM, D), lambda i: (i, 0)),
        pl.BlockSpec((1, D), lambda i: (0, 0)),
        pl.BlockSpec((1, D), lambda i: (0, 0)),
    ],
    out_specs=pl.BlockSpec((BM, D), lambda i: (i, 0)),
    out_shape=jax.ShapeDtypeStruct((N, D), jnp.float32),
)


def kernel(x, edge_index, adj_vals, W1, a1, W2, ln_w, ln_b):
    pad = EP - E
    rowp = jnp.concatenate([edge_index[0], jnp.zeros((pad,), jnp.int32)])
    colp = jnp.concatenate([edge_index[1], jnp.zeros((pad,), jnp.int32)])
    row2d = rowp.reshape(EP // SUB, SUB)
    col2d = colp.reshape(EP // SUB, SUB)
    row2d_64 = rowp.reshape(EP // S2, S2)
    col2d_64 = colp.reshape(EP // S2, S2)
    adjp = jnp.concatenate([adj_vals, jnp.zeros((pad,), jnp.float32)])
    a2 = jnp.concatenate([a1[:D], a1[D:]], axis=1)
    a2 = jnp.pad(a2, ((0, 0), (0, 6)))

    h1, s2 = _mm1(x, W1, a2)
    stop = s2[:, 0]
    sbot = s2[:, 1]

    part1, w = _sc_attend_agg(row2d, col2d, adjp, h1, stop, sbot)
    h2 = _mm2(part1, W2)
    part2 = _sc_agg(row2d, col2d, w, h2)
    return _final(part2, x, ln_w.reshape(1, D), ln_b.reshape(1, D))


# speedup vs baseline: 1.4563x; 1.0003x over previous
import functools

import jax
import jax.numpy as jnp
from jax import lax
from jax.experimental import pallas as pl
from jax.experimental.pallas import tpu as pltpu
from jax.experimental.pallas import tpu_sc as plsc

N = 10000
E = 320000
D = 128

NC = 2
NS = 16
L = 16
NW = NC * NS
FH = D // NC
SUB = 128
IDR = 8
CHE = IDR * SUB
NCH = 20
EPT = CHE * NCH
IRT = EPT // SUB
EP = EPT * NS
NP = 10240
RPT = NP // NS
BM = 1000
SUBP = 128


_mesh = plsc.VectorSubcoreMesh(core_axis_name="c", subcore_axis_name="s")
_sc_params = pltpu.CompilerParams(needs_layout_passes=False,
                                  use_tc_tiling_on_sc=False)


def _zero_acc(zb, acc_sh, sid):
    @pl.loop(0, SUB)
    def _z(i):
        for j in range(FH // L):
            zb[i, pl.ds(j * L, L)] = jnp.zeros((L,), jnp.float32)

    for kk in range(RPT // SUB):
        pltpu.sync_copy(zb.at[pl.ds(0, SUB)],
                        acc_sh.at[pl.ds(sid * RPT + kk * SUB, SUB)])


NB = 4


def _sc_body(row_hbm, col_hbm, hsrc, out_hbm,
             rowi_v, coli_v, wvs, rows_v, acc_sh, isem, gsem, ssem,
             sid, cid, pre_fn, weight_fn, tail_fn, nb=NB,
             sub=SUB, idr=IDR, irt=IRT):
    def idx_start(c, b):
        r0 = sid * irt + c * idr
        pltpu.async_copy(row_hbm.at[pl.ds(r0, idr)], rowi_v.at[b],
                         isem.at[b])
        pltpu.async_copy(col_hbm.at[pl.ds(r0, idr)], coli_v.at[b],
                         isem.at[b])

    def idx_wait(c, b):
        r0 = sid * irt + c * idr
        pltpu.make_async_copy(
            row_hbm.at[pl.ds(r0, idr)], rowi_v.at[b], isem.at[b]).wait()
        pltpu.make_async_copy(
            col_hbm.at[pl.ds(r0, idr)], coli_v.at[b], isem.at[b]).wait()

    def gather_start(cb, s, b):
        pltpu.async_copy(hsrc.at[coli_v.at[cb, s]], rows_v.at[b], gsem.at[b])

    def gather_wait(cb, s, b):
        pltpu.make_async_copy(
            hsrc.at[coli_v.at[cb, s]], rows_v.at[b], gsem.at[b]).wait()

    def scatter_start(cb, s, b):
        pltpu.async_copy(rows_v.at[b], acc_sh.at[rowi_v.at[cb, s]],
                         ssem.at[b], add=True)

    def scatter_drain(b):
        pltpu.make_async_copy(rows_v.at[b], acc_sh.at[pl.ds(0, sub)],
                              ssem.at[b]).wait()

    def chunk(c, cb):
        pre_fn(c, cb)

        @pl.when(c + 1 < NCH)
        def _():
            idx_start(c + 1, 1 - cb)

        for s in range(idr):
            b = s % nb
            ahead = s + nb - 1
            ba = ahead % nb
            if ahead < idr:
                if s == 0:
                    @pl.when(c > 0)
                    def _():
                        scatter_drain(ba)
                else:
                    scatter_drain(ba)
                gather_start(cb, ahead, ba)
            else:
                if ahead == idr:
                    @pl.when(c + 1 < NCH)
                    def _():
                        idx_wait(c + 1, 1 - cb)

                @pl.when(c + 1 < NCH)
                def _():
                    scatter_drain(ba)
                    gather_start(1 - cb, ahead - idr, ba)
            weight_fn(cb, s)
            gather_wait(cb, s, b)
            rows_b = rows_v.at[b]

            @pl.loop(0, sub, unroll=4)
            def _scale(e, s=s, wv=wvs[cb], rows_b=rows_b):
                wb = plsc.load_gather(
                    wv, [jnp.zeros((L,), jnp.int32) + (s * sub + e)])
                for j in range(FH // L):
                    sl = pl.ds(j * L, L)
                    rows_b[e, sl] = rows_b[e, sl] * wb

            scatter_start(cb, s, b)
        tail_fn(c, cb)

    idx_start(0, 0)
    idx_wait(0, 0)
    for s0 in range(nb - 1):
        gather_start(0, s0, s0)

    @pl.loop(0, NCH, step=2)
    def _main(c):
        chunk(c, 0)
        chunk(c + 1, 1)

    for b in range(nb):
        scatter_drain(b)

    plsc.subcore_barrier()
    pltpu.sync_copy(acc_sh.at[pl.ds(sid * RPT, RPT)],
                    out_hbm.at[cid, pl.ds(sid * RPT, RPT)])


@functools.partial(
    pl.kernel,
    out_type=(
        jax.ShapeDtypeStruct((NC, NP, FH), jnp.float32),
        jax.ShapeDtypeStruct((EP,), jnp.float32),
    ),
    mesh=_mesh,
    compiler_params=_sc_params,
    scratch_types=[
        pltpu.VMEM((2, IDR, SUB), jnp.int32),
        pltpu.VMEM((2, IDR, SUB), jnp.int32),
        pltpu.VMEM((CHE,), jnp.float32),
        pltpu.VMEM((CHE,), jnp.float32),
        pltpu.VMEM((CHE,), jnp.float32),
        pltpu.VMEM((CHE,), jnp.float32),
        pltpu.VMEM((2, SUB, FH), jnp.float32),
        pltpu.VMEM((N,), jnp.float32),
        pltpu.VMEM((N,), jnp.float32),
        pltpu.VMEM_SHARED((NP, FH), jnp.float32),
        pltpu.VMEM_SHARED((N, FH), jnp.float32),
        pltpu.SemaphoreType.DMA((2,)),
        pltpu.SemaphoreType.DMA((2,)),
        pltpu.SemaphoreType.DMA((2,)),
        pltpu.SemaphoreType.DMA((2,)),
    ],
)
def _sc_attend_agg(row_hbm, col_hbm, adj_hbm, h_hbm, stop_hbm, sbot_hbm,
                   out_hbm, w_hbm,
                   rowi_v, coli_v, adj0_v, adj1_v, w0_v, w1_v, rows_v,
                   stop_v, sbot_v,
                   acc_sh, h_sh, isem, gsem, asem, ssem):
    cid = lax.axis_index("c")
    sid = lax.axis_index("s")
    advs = (adj0_v, adj1_v)
    wvs = (w0_v, w1_v)

    pltpu.sync_copy(stop_hbm, stop_v)
    pltpu.sync_copy(sbot_hbm, sbot_v)
    _zero_acc(rows_v.at[0], acc_sh, sid)
    pltpu.sync_copy(h_hbm.at[cid, pl.ds(sid * (N // NS), N // NS)],
                    h_sh.at[pl.ds(sid * (N // NS), N // NS)])
    plsc.subcore_barrier()

    def adj_start(c, b):
        e0 = sid * EPT + c * CHE
        pltpu.async_copy(adj_hbm.at[pl.ds(e0, CHE)], advs[b], asem.at[b])

    def pre_fn(c, cb):
        e0 = sid * EPT + c * CHE
        pltpu.make_async_copy(
            adj_hbm.at[pl.ds(e0, CHE)], advs[cb], asem.at[cb]).wait()

    def weight_fn(cb, s):
        @pl.loop(0, SUB // L, unroll=2)
        def _wg(g, cb=cb, s=s):
            o = g * L
            r = rowi_v[cb, s, pl.ds(o, L)]
            cc = coli_v[cb, s, pl.ds(o, L)]
            t = (plsc.load_gather(stop_v, [r])
                 + plsc.load_gather(sbot_v, [cc]))
            t = jnp.where(t >= 0.0, t, 0.2 * t)
            w = 1.0 / (1.0 + jnp.exp(-t))
            wvs[cb][pl.ds(s * SUB + o, L)] = (
                w * advs[cb][pl.ds(s * SUB + o, L)])

    def tail_fn(c, cb):
        @pl.when(c + 1 < NCH)
        def _():
            adj_start(c + 1, 1 - cb)

        @pl.when(cid == 0)
        def _():
            e0 = sid * EPT + c * CHE
            pltpu.sync_copy(wvs[cb], w_hbm.at[pl.ds(e0, CHE)])

    adj_start(0, 0)
    _sc_body(row_hbm, col_hbm, h_sh, out_hbm,
             rowi_v, coli_v, wvs, rows_v, acc_sh, isem, gsem, ssem,
             sid, cid, pre_fn, weight_fn, tail_fn, nb=2)


@functools.partial(
    pl.kernel,
    out_type=jax.ShapeDtypeStruct((NC, NP, FH), jnp.float32),
    mesh=_mesh,
    compiler_params=_sc_params,
    scratch_types=[
        pltpu.VMEM((2, CHE // SUBP, SUBP), jnp.int32),
        pltpu.VMEM((2, CHE // SUBP, SUBP), jnp.int32),
        pltpu.VMEM((CHE,), jnp.float32),
        pltpu.VMEM((CHE,), jnp.float32),
        pltpu.VMEM((NB, SUBP, FH), jnp.float32),
        pltpu.VMEM_SHARED((NP, FH), jnp.float32),
        pltpu.VMEM_SHARED((N, FH), jnp.float32),
        pltpu.SemaphoreType.DMA((2,)),
        pltpu.SemaphoreType.DMA((NB,)),
        pltpu.SemaphoreType.DMA((2,)),
        pltpu.SemaphoreType.DMA((NB,)),
    ],
)
def _sc_agg(row_hbm, col_hbm, w_hbm, h_hbm,
            out_hbm,
            rowi_v, coli_v, w0_v, w1_v, rows_v, acc_sh, h_sh,
            isem, gsem, asem, ssem):
    cid = lax.axis_index("c")
    sid = lax.axis_index("s")
    wvs = (w0_v, w1_v)

    _zero_acc(rows_v.at[0], acc_sh, sid)
    pltpu.sync_copy(h_hbm.at[cid, pl.ds(sid * (N // NS), N // NS)],
                    h_sh.at[pl.ds(sid * (N // NS), N // NS)])
    plsc.subcore_barrier()

    def w_start(c, b):
        e0 = sid * EPT + c * CHE
        pltpu.async_copy(w_hbm.at[pl.ds(e0, CHE)], wvs[b], asem.at[b])

    def pre_fn(c, cb):
        e0 = sid * EPT + c * CHE
        pltpu.make_async_copy(
            w_hbm.at[pl.ds(e0, CHE)], wvs[cb], asem.at[cb]).wait()

    def weight_fn(cb, s):
        del cb, s

    def tail_fn(c, cb):
        @pl.when(c + 1 < NCH)
        def _():
            w_start(c + 1, 1 - cb)

    w_start(0, 0)
    _sc_body(row_hbm, col_hbm, h_sh, out_hbm,
             rowi_v, coli_v, wvs, rows_v, acc_sh, isem, gsem, ssem,
             sid, cid, pre_fn, weight_fn, tail_fn, nb=NB,
             sub=SUBP, idr=CHE // SUBP, irt=EPT // SUBP)


def _mm1_body(x_ref, w1_ref, a2_ref, h1_ref, s2_ref):
    h1 = jnp.dot(x_ref[...], w1_ref[...], preferred_element_type=jnp.float32)
    h1_ref[0] = h1[:, :FH]
    h1_ref[1] = h1[:, FH:]
    s2_ref[...] = jnp.dot(h1, a2_ref[...], preferred_element_type=jnp.float32)


_mm1 = pl.pallas_call(
    _mm1_body,
    grid=(N // BM,),
    in_specs=[
        pl.BlockSpec((BM, D), lambda i: (i, 0)),
        pl.BlockSpec((D, D), lambda i: (0, 0)),
        pl.BlockSpec((D, 8), lambda i: (0, 0)),
    ],
    out_specs=[
        pl.BlockSpec((NC, BM, FH), lambda i: (0, i, 0)),
        pl.BlockSpec((BM, 8), lambda i: (i, 0)),
    ],
    out_shape=[
        jax.ShapeDtypeStruct((NC, N, FH), jnp.float32),
        jax.ShapeDtypeStruct((N, 8), jnp.float32),
    ],
)


def _mm2_body(p_ref, w2_ref, h2_ref):
    h = jnp.maximum(jnp.concatenate([p_ref[0], p_ref[1]], axis=1), 0.0)
    h2 = jnp.dot(h, w2_ref[...], preferred_element_type=jnp.float32)
    h2_ref[0] = h2[:, :FH]
    h2_ref[1] = h2[:, FH:]


_mm2 = pl.pallas_call(
    _mm2_body,
    grid=(N // BM,),
    in_specs=[
        pl.BlockSpec((NC, BM, FH), lambda i: (0, i, 0)),
        pl.BlockSpec((D, D), lambda i: (0, 0)),
    ],
    out_specs=pl.BlockSpec((NC, BM, FH), lambda i: (0, i, 0)),
    out_shape=jax.ShapeDtypeStruct((NC, N, FH), jnp.float32),
)


def _final_body(p_ref, x_ref, lnw_ref, lnb_ref, o_ref):
    h = jnp.maximum(jnp.concatenate([p_ref[0], p_ref[1]], axis=1), 0.0)
    h = h + x_ref[...]
    mean = jnp.mean(h, axis=1, keepdims=True)
    d = h - mean
    var = jnp.mean(d * d, axis=1, keepdims=True)
    o_ref[...] = d * lax.rsqrt(var + 1e-5) * lnw_ref[...] + lnb_ref[...]


_final = pl.pallas_call(
    _final_body,
    grid=(N // BM,),
    in_specs=[
        pl.BlockSpec((NC, BM, FH), lambda i: (0, i, 0)),
        pl.BlockSpec((BM, D), lambda i: (i, 0)),
        pl.BlockSpec((1, D), lambda i: (0, 0)),
        pl.BlockSpec((1, D), lambda i: (0, 0)),
    ],
    out_specs=pl.BlockSpec((BM, D), lambda i: (i, 0)),
    out_shape=jax.ShapeDtypeStruct((N, D), jnp.float32),
)


def kernel(x, edge_index, adj_vals, W1, a1, W2, ln_w, ln_b):
    pad = EP - E
    rowp = jnp.concatenate([edge_index[0], jnp.zeros((pad,), jnp.int32)])
    colp = jnp.concatenate([edge_index[1], jnp.zeros((pad,), jnp.int32)])
    row2d = rowp.reshape(EP // SUB, SUB)
    col2d = colp.reshape(EP // SUB, SUB)
    row2dp = rowp.reshape(EP // SUBP, SUBP)
    col2dp = colp.reshape(EP // SUBP, SUBP)
    adjp = jnp.concatenate([adj_vals, jnp.zeros((pad,), jnp.float32)])
    a2 = jnp.concatenate([a1[:D], a1[D:]], axis=1)
    a2 = jnp.pad(a2, ((0, 0), (0, 6)))

    h1, s2 = _mm1(x, W1, a2)
    stop = s2[:, 0]
    sbot = s2[:, 1]

    part1, w = _sc_attend_agg(row2d, col2d, adjp, h1, stop, sbot)
    h2 = _mm2(part1, W2)
    part2 = _sc_agg(row2dp, col2dp, w, h2)
    return _final(part2, x, ln_w.reshape(1, D), ln_b.reshape(1, D))
